# baseline (speedup 1.0000x reference)
import jax
import jax.numpy as jnp
from jax import lax
from jax.experimental import pallas as pl
from jax.experimental.pallas import tpu as pltpu
from jax.experimental.pallas import tpu_sc as plsc

L_E = 8192
D = 1152
NLANE = 16
NTILE = 32
ROWS_PER_TILE = L_E // NTILE
CHUNK = 32
NCHUNK = ROWS_PER_TILE // CHUNK
RBLK = 8
NBLK = CHUNK // RBLK
KCH = D // NLANE


def _rsqrt32(x):
    i = lax.bitcast_convert_type(x, jnp.int32)
    i = jnp.int32(0x5F3759DF) - lax.shift_right_logical(i, 1)
    y = lax.bitcast_convert_type(i, jnp.float32)
    half = jnp.float32(0.5)
    three_half = jnp.float32(1.5)
    for _ in range(3):
        y = y * (three_half - half * x * y * y)
    return y


def _sc_scan_body(mt_hbm, mem_hbm, out_hbm, csim_hbm, cidx_hbm,
                  mt_v, buf_v, csim_v, cidx_v):
    c = lax.axis_index("c")
    s = lax.axis_index("s")
    wid = c * 16 + s
    base_row = wid * ROWS_PER_TILE
    pltpu.sync_copy(mt_hbm, mt_v)

    def chunk_body(ch, best):
        best_sim, best_idx = best
        row0 = base_row + ch * CHUNK
        pltpu.sync_copy(mem_hbm.at[pl.ds(row0, CHUNK)], buf_v)
        for blk in range(NBLK):
            r0 = blk * RBLK

            def kbody(k, accs):
                dacc, sacc = accs
                mtk = mt_v[pl.ds(k * NLANE, NLANE)]
                nd, ns = [], []
                for r in range(RBLK):
                    v = buf_v[r0 + r, pl.ds(k * NLANE, NLANE)]
                    nd.append(dacc[r] + v * mtk)
                    ns.append(sacc[r] + v * v)
                return tuple(nd), tuple(ns)

            zeros = tuple(jnp.zeros((NLANE,), jnp.float32)
                          for _ in range(RBLK))
            dvecs, svecs = lax.fori_loop(0, KCH, kbody, (zeros, zeros))
            for r in range(RBLK):
                dsum = jnp.sum(dvecs[r])
                ssum = jnp.sum(svecs[r])
                sim = dsum * _rsqrt32(ssum)
                ridx = row0 + r0 + r
                take = sim > best_sim
                best_sim = jnp.where(take, sim, best_sim)
                best_idx = jnp.where(take, ridx, best_idx)
        pltpu.sync_copy(buf_v, out_hbm.at[pl.ds(row0, CHUNK)])
        return best_sim, best_idx

    init = (jnp.float32(-jnp.inf), jnp.int32(0))
    best_sim, best_idx = lax.fori_loop(0, NCHUNK, chunk_body, init)

    lane = lax.iota(jnp.int32, NLANE)
    csim_v[...] = jnp.where(lane == 0, best_sim, jnp.float32(-jnp.inf))
    cidx_v[...] = jnp.where(lane == 0, best_idx, jnp.int32(2**30))
    pltpu.sync_copy(csim_v, csim_hbm.at[wid])
    pltpu.sync_copy(cidx_v, cidx_hbm.at[wid])


_sc_scan = pl.kernel(
    _sc_scan_body,
    out_type=(
        jax.ShapeDtypeStruct((L_E, D), jnp.float32),
        jax.ShapeDtypeStruct((NTILE, NLANE), jnp.float32),
        jax.ShapeDtypeStruct((NTILE, NLANE), jnp.int32),
    ),
    mesh=plsc.VectorSubcoreMesh(core_axis_name="c", subcore_axis_name="s"),
    compiler_params=pltpu.CompilerParams(needs_layout_passes=False),
    scratch_types=[
        pltpu.VMEM((D,), jnp.float32),
        pltpu.VMEM((CHUNK, D), jnp.float32),
        pltpu.VMEM((NLANE,), jnp.float32),
        pltpu.VMEM((NLANE,), jnp.int32),
    ],
)


def _fixup_body(sim_ref, idx_ref, mt_ref, src_ref, out_ref, sem):
    del src_ref
    sims = sim_ref[...]
    idxs = idx_ref[...]
    m = jnp.max(sims)
    winner = jnp.min(jnp.where(sims == m, idxs, jnp.int32(2**30)))
    cp = pltpu.make_async_copy(mt_ref, out_ref.at[pl.ds(winner, 1)], sem)
    cp.start()
    cp.wait()


def kernel(M_t, memory):
    copied, csim, cidx = _sc_scan(M_t, memory)
    out = pl.pallas_call(
        _fixup_body,
        out_shape=jax.ShapeDtypeStruct((L_E, D), jnp.float32),
        in_specs=[
            pl.BlockSpec(memory_space=pltpu.VMEM),
            pl.BlockSpec(memory_space=pltpu.VMEM),
            pl.BlockSpec(memory_space=pltpu.VMEM),
            pl.BlockSpec(memory_space=pl.ANY),
        ],
        out_specs=pl.BlockSpec(memory_space=pl.ANY),
        scratch_shapes=[pltpu.SemaphoreType.DMA],
        input_output_aliases={3: 0},
    )(csim.reshape(4, 128), cidx.reshape(4, 128), M_t.reshape(1, D), copied)
    return out


# speedup vs baseline: 1.3351x; 1.3351x over previous
import jax
import jax.numpy as jnp
from jax import lax
from jax.experimental import pallas as pl
from jax.experimental.pallas import tpu as pltpu
from jax.experimental.pallas import tpu_sc as plsc

L_E = 8192
D = 1152
NLANE = 16
NTILE = 32
ROWS_PER_TILE = L_E // NTILE
CHUNK = 16
NCHUNK = ROWS_PER_TILE // CHUNK
NBUF = 4
KCH = D // NLANE


def _rsqrt32(x):
    i = lax.bitcast_convert_type(x, jnp.int32)
    i = jnp.int32(0x5F3759DF) - lax.shift_right_logical(i, 1)
    y = lax.bitcast_convert_type(i, jnp.float32)
    half = jnp.float32(0.5)
    three_half = jnp.float32(1.5)
    for _ in range(3):
        y = y * (three_half - half * x * y * y)
    return y


def _sc_scan_body(mt_hbm, mem_hbm, out_hbm, csim_hbm, cidx_hbm,
                  mt_v, b0, b1, b2, b3, csim_v, cidx_v,
                  si0, si1, si2, si3, so0, so1, so2, so3):
    bufs = (b0, b1, b2, b3)
    isems = (si0, si1, si2, si3)
    osems = (so0, so1, so2, so3)
    c = lax.axis_index("c")
    s = lax.axis_index("s")
    wid = c * 16 + s
    base_row = wid * ROWS_PER_TILE
    pltpu.sync_copy(mt_hbm, mt_v)

    def in_cp(ch, b):
        return pltpu.make_async_copy(
            mem_hbm.at[pl.ds(base_row + ch * CHUNK, CHUNK)], bufs[b],
            isems[b])

    def out_cp(ch, b):
        return pltpu.make_async_copy(
            bufs[b], out_hbm.at[pl.ds(base_row + ch * CHUNK, CHUNK)],
            osems[b])

    in_cp(0, 0).start()
    in_cp(1, 1).start()

    def compute(buf, row0, best_sim, best_idx):
        def kbody(k, accs):
            dacc, sacc = accs
            mtk = mt_v[pl.ds(k * NLANE, NLANE)]
            nd, ns = [], []
            for r in range(CHUNK):
                v = buf[r, pl.ds(k * NLANE, NLANE)]
                nd.append(dacc[r] + v * mtk)
                ns.append(sacc[r] + v * v)
            return tuple(nd), tuple(ns)

        zeros = tuple(jnp.zeros((NLANE,), jnp.float32)
                      for _ in range(CHUNK))
        dvecs, svecs = lax.fori_loop(0, KCH, kbody, (zeros, zeros))
        for r in range(CHUNK):
            dsum = jnp.sum(dvecs[r])
            ssum = jnp.sum(svecs[r])
            sim = dsum * _rsqrt32(ssum)
            ridx = row0 + r
            take = sim > best_sim
            best_sim = jnp.where(take, sim, best_sim)
            best_idx = jnp.where(take, ridx, best_idx)
        return best_sim, best_idx

    def quad(q, best):
        best_sim, best_idx = best
        for j in range(NBUF):
            ch = NBUF * q + j
            in_cp(ch, j).wait()
            best_sim, best_idx = compute(
                bufs[j], base_row + ch * CHUNK, best_sim, best_idx)
            out_cp(ch, j).start()
            nxt = ch + 2
            bb = (j + 2) % NBUF

            @pl.when(nxt < NCHUNK)
            def _():
                @pl.when(ch >= 2)
                def _():
                    out_cp(ch - 2, bb).wait()
                in_cp(nxt, bb).start()
        return best_sim, best_idx

    init = (jnp.float32(-jnp.inf), jnp.int32(0))
    best_sim, best_idx = lax.fori_loop(0, NCHUNK // NBUF, quad, init)

    for b in range(NBUF):
        out_cp(NCHUNK - NBUF + b, b).wait()

    lane = lax.iota(jnp.int32, NLANE)
    csim_v[...] = jnp.where(lane == 0, best_sim, jnp.float32(-jnp.inf))
    cidx_v[...] = jnp.where(lane == 0, best_idx, jnp.int32(2**30))
    pltpu.sync_copy(csim_v, csim_hbm.at[wid])
    pltpu.sync_copy(cidx_v, cidx_hbm.at[wid])


_sc_scan = pl.kernel(
    _sc_scan_body,
    out_type=(
        jax.ShapeDtypeStruct((L_E, D), jnp.float32),
        jax.ShapeDtypeStruct((NTILE, NLANE), jnp.float32),
        jax.ShapeDtypeStruct((NTILE, NLANE), jnp.int32),
    ),
    mesh=plsc.VectorSubcoreMesh(core_axis_name="c", subcore_axis_name="s"),
    compiler_params=pltpu.CompilerParams(needs_layout_passes=False),
    scratch_types=[
        pltpu.VMEM((D,), jnp.float32),
        pltpu.VMEM((CHUNK, D), jnp.float32),
        pltpu.VMEM((CHUNK, D), jnp.float32),
        pltpu.VMEM((CHUNK, D), jnp.float32),
        pltpu.VMEM((CHUNK, D), jnp.float32),
        pltpu.VMEM((NLANE,), jnp.float32),
        pltpu.VMEM((NLANE,), jnp.int32),
        pltpu.SemaphoreType.DMA,
        pltpu.SemaphoreType.DMA,
        pltpu.SemaphoreType.DMA,
        pltpu.SemaphoreType.DMA,
        pltpu.SemaphoreType.DMA,
        pltpu.SemaphoreType.DMA,
        pltpu.SemaphoreType.DMA,
        pltpu.SemaphoreType.DMA,
    ],
)


def _fixup_body(sim_ref, idx_ref, mt_ref, src_ref, out_ref, sem):
    del src_ref
    sims = sim_ref[...]
    idxs = idx_ref[...]
    m = jnp.max(sims)
    winner = jnp.min(jnp.where(sims == m, idxs, jnp.int32(2**30)))
    cp = pltpu.make_async_copy(mt_ref, out_ref.at[pl.ds(winner, 1)], sem)
    cp.start()
    cp.wait()


def kernel(M_t, memory):
    copied, csim, cidx = _sc_scan(M_t, memory)
    out = pl.pallas_call(
        _fixup_body,
        out_shape=jax.ShapeDtypeStruct((L_E, D), jnp.float32),
        in_specs=[
            pl.BlockSpec(memory_space=pltpu.VMEM),
            pl.BlockSpec(memory_space=pltpu.VMEM),
            pl.BlockSpec(memory_space=pltpu.VMEM),
            pl.BlockSpec(memory_space=pl.ANY),
        ],
        out_specs=pl.BlockSpec(memory_space=pl.ANY),
        scratch_shapes=[pltpu.SemaphoreType.DMA],
        input_output_aliases={3: 0},
    )(csim.reshape(4, 128), cidx.reshape(4, 128), M_t.reshape(1, D), copied)
    return out
